# Initial kernel scaffold; baseline (speedup 1.0000x reference)
import jax
import jax.numpy as jnp
from jax.experimental import pallas as pl


def kernel(x, edge_attr, W_g, gru_Wih, gru_Whh, gru_bih, gru_bhh, g_W1, g_b1, g_W2, g_b2, g_W3, g_b3, ai_W1, ai_b1, ai_W2, ai_b2, aj_W, aj_b, o_W1, o_b1, o_W2, o_b2, o_W3, o_b3, edge_index, batch):
    raise NotImplementedError("write your pallas kernel here")


# speedup vs baseline: 7.1621x; 7.1621x over previous
import functools

import jax
import jax.numpy as jnp
from jax import lax
from jax.experimental import pallas as pl
from jax.experimental.pallas import tpu as pltpu
from jax.experimental.pallas import tpu_sc as plsc

N = 10000
D = 128
NG = 64
CUTOFF = 3.5
LAYERS = 4

NC = 2
NS = 16
NW = NC * NS

KB = 128
NBLK = 81
EPW = NBLK * KB
E_PAD = NW * EPW
NP = 10240
NPS = NP // NS

_f32 = jnp.float32
_i32 = jnp.int32


def _ss(v):
    return v / (1.0 + jnp.abs(v))


RB = 1024


def _dense0_body(h_ref, Wg_ref, m_ref):
    m_ref[...] = jnp.dot(h_ref[...], Wg_ref[...], preferred_element_type=_f32)


def _tc_dense0(h, Wg):
    return pl.pallas_call(
        _dense0_body,
        grid=(NP // RB,),
        in_specs=[
            pl.BlockSpec((RB, D), lambda i: (i, 0)),
            pl.BlockSpec((D, D), lambda i: (0, 0)),
        ],
        out_specs=pl.BlockSpec((RB, D), lambda i: (i, 0)),
        out_shape=jax.ShapeDtypeStruct((NP, D), _f32),
    )(h, Wg)


def _gru_body(a0_ref, a1_ref, den_ref, h_ref, Wih_ref, Whh_ref, bih_ref,
              bhh_ref, Wg_ref, *out_refs, with_dense):
    den = den_ref[...]
    rden = 1.0 / (den[:, 0:1] + den[:, 1:2] + 1e-16)
    agg = (a0_ref[...] + a1_ref[...]) * rden
    h = h_ref[...]
    gi = jnp.dot(agg, Wih_ref[...].T, preferred_element_type=_f32) + bih_ref[...]
    gh = jnp.dot(h, Whh_ref[...].T, preferred_element_type=_f32) + bhh_ref[...]
    r = jax.nn.sigmoid(gi[:, :D] + gh[:, :D])
    z = jax.nn.sigmoid(gi[:, D:2 * D] + gh[:, D:2 * D])
    n = jnp.tanh(gi[:, 2 * D:] + r * gh[:, 2 * D:])
    h2 = (1.0 - z) * n + z * h
    out_refs[0][...] = h2
    if with_dense:
        out_refs[1][...] = jnp.dot(h2, Wg_ref[...],
                                   preferred_element_type=_f32)


def _tc_gru(a0, a1, denT, h, Wih, Whh, bih, bhh, Wg, with_dense):
    out_specs = [pl.BlockSpec((RB, D), lambda i: (i, 0))]
    out_shape = [jax.ShapeDtypeStruct((NP, D), _f32)]
    if with_dense:
        out_specs.append(pl.BlockSpec((RB, D), lambda i: (i, 0)))
        out_shape.append(jax.ShapeDtypeStruct((NP, D), _f32))
    return pl.pallas_call(
        functools.partial(_gru_body, with_dense=with_dense),
        grid=(NP // RB,),
        in_specs=[
            pl.BlockSpec((RB, D), lambda i: (i, 0)),
            pl.BlockSpec((RB, D), lambda i: (i, 0)),
            pl.BlockSpec((RB, NC), lambda i: (i, 0)),
            pl.BlockSpec((RB, D), lambda i: (i, 0)),
            pl.BlockSpec((3 * D, D), lambda i: (0, 0)),
            pl.BlockSpec((3 * D, D), lambda i: (0, 0)),
            pl.BlockSpec((1, 3 * D), lambda i: (0, 0)),
            pl.BlockSpec((1, 3 * D), lambda i: (0, 0)),
            pl.BlockSpec((D, D), lambda i: (0, 0)),
        ],
        out_specs=out_specs,
        out_shape=out_shape,
    )(a0, a1, denT, h, Wih, Whh, bih.reshape(1, -1), bhh.reshape(1, -1), Wg)


EB = 4096


def _gate_body(msg_ref, ea_ref, W1_ref, b1_ref, W2_ref, b2_ref, w3_ref,
               b3_ref, gexp_ref, q_ref):
    msg = msg_ref[...]
    ea = ea_ref[...]
    mask = ea <= CUTOFF
    ew = jnp.where(mask, ea, 0.0)
    g1 = jnp.dot(msg, W1_ref[...].T, preferred_element_type=_f32)
    t1 = _ss(ew * g1 + b1_ref[...])
    t2 = _ss(jnp.dot(t1, W2_ref[...].T, preferred_element_type=_f32)
             + b2_ref[...])
    gate = jnp.sum(t2 * w3_ref[...], axis=1, keepdims=True) + b3_ref[0, 0]
    gexp = jnp.where(mask, jnp.exp(gate), 0.0)
    gexp_ref[...] = gexp
    q_ref[...] = gexp * ew


def _tc_gate(msg, ea_col, g_W1, g_b1, g_W2, g_b2, g_W3, g_b3):
    return pl.pallas_call(
        _gate_body,
        grid=(E_PAD // EB,),
        in_specs=[
            pl.BlockSpec((EB, D), lambda i: (i, 0)),
            pl.BlockSpec((EB, 1), lambda i: (i, 0)),
            pl.BlockSpec((64, D), lambda i: (0, 0)),
            pl.BlockSpec((1, 64), lambda i: (0, 0)),
            pl.BlockSpec((32, 64), lambda i: (0, 0)),
            pl.BlockSpec((1, 32), lambda i: (0, 0)),
            pl.BlockSpec((1, 32), lambda i: (0, 0)),
            pl.BlockSpec((1, 1), lambda i: (0, 0)),
        ],
        out_specs=[
            pl.BlockSpec((EB, 1), lambda i: (i, 0)),
            pl.BlockSpec((EB, 1), lambda i: (i, 0)),
        ],
        out_shape=[
            jax.ShapeDtypeStruct((E_PAD, 1), _f32),
            jax.ShapeDtypeStruct((E_PAD, 1), _f32),
        ],
    )(msg, ea_col, g_W1, g_b1.reshape(1, 64), g_W2, g_b2.reshape(1, 32),
      g_W3.reshape(1, 32), g_b3.reshape(1, 1))


def _readout_body(h_ref, x_ref, bf_ref, W1h_ref, W1x_ref, b1_ref, W2_ref,
                  b2_ref, Wj_ref, bj_ref, oW1_ref, ob1_ref, oW2_ref, ob2_ref,
                  oW3_ref, ob3_ref, out_ref, acc_ref):
    i = pl.program_id(0)

    @pl.when(i == 0)
    def _():
        acc_ref[...] = jnp.zeros((NG, D), _f32)

    h = h_ref[...]
    x = x_ref[...]
    a1 = _ss(jnp.dot(h, W1h_ref[...].T, preferred_element_type=_f32)
             + jnp.dot(x, W1x_ref[...].T, preferred_element_type=_f32)
             + b1_ref[...])
    ai = _ss(jnp.dot(a1, W2_ref[...].T, preferred_element_type=_f32)
             + b2_ref[...])
    aj = _ss(jnp.dot(x, Wj_ref[...].T, preferred_element_type=_f32)
             + bj_ref[...])
    rowmax = jnp.max(ai, axis=1, keepdims=True)
    e = jnp.exp(ai - rowmax)
    attn = e / jnp.sum(e, axis=1, keepdims=True) * aj
    gids = lax.broadcasted_iota(_i32, (1, NG), 1).astype(_f32)
    oh = (bf_ref[...] == gids).astype(_f32)
    acc_ref[...] += lax.dot_general(oh, attn, (((0,), (0,)), ((), ())),
                                    preferred_element_type=_f32)

    @pl.when(i == pl.num_programs(0) - 1)
    def _():
        pooled = acc_ref[...]
        y = jax.nn.relu(jnp.dot(pooled, oW1_ref[...].T,
                                preferred_element_type=_f32) + ob1_ref[...])
        y = jax.nn.relu(jnp.dot(y, oW2_ref[...].T,
                                preferred_element_type=_f32) + ob2_ref[...])
        out_ref[...] = (jnp.sum(y * oW3_ref[...], axis=1, keepdims=True)
                        + ob3_ref[0, 0])


def _tc_readout(h, x, batchf, ai_W1, ai_b1, ai_W2, ai_b2, aj_W, aj_b,
                o_W1, o_b1, o_W2, o_b2, o_W3, o_b3):
    H1, H2 = o_W1.shape[0], o_W2.shape[0]
    return pl.pallas_call(
        _readout_body,
        grid=(NP // RB,),
        in_specs=[
            pl.BlockSpec((RB, D), lambda i: (i, 0)),
            pl.BlockSpec((RB, D), lambda i: (i, 0)),
            pl.BlockSpec((RB, 1), lambda i: (i, 0)),
            pl.BlockSpec((D, D), lambda i: (0, 0)),
            pl.BlockSpec((D, D), lambda i: (0, 0)),
            pl.BlockSpec((1, D), lambda i: (0, 0)),
            pl.BlockSpec((D, D), lambda i: (0, 0)),
            pl.BlockSpec((1, D), lambda i: (0, 0)),
            pl.BlockSpec((D, D), lambda i: (0, 0)),
            pl.BlockSpec((1, D), lambda i: (0, 0)),
            pl.BlockSpec((H1, D), lambda i: (0, 0)),
            pl.BlockSpec((1, H1), lambda i: (0, 0)),
            pl.BlockSpec((H2, H1), lambda i: (0, 0)),
            pl.BlockSpec((1, H2), lambda i: (0, 0)),
            pl.BlockSpec((1, H2), lambda i: (0, 0)),
            pl.BlockSpec((1, 1), lambda i: (0, 0)),
        ],
        out_specs=pl.BlockSpec((NG, 1), lambda i: (0, 0)),
        out_shape=jax.ShapeDtypeStruct((NG, 1), _f32),
        scratch_shapes=[pltpu.VMEM((NG, D), _f32)],
    )(h, x, batchf, ai_W1[:, :D], ai_W1[:, D:], ai_b1.reshape(1, D),
      ai_W2, ai_b2.reshape(1, D), aj_W, aj_b.reshape(1, D),
      o_W1, o_b1.reshape(1, H1), o_W2, o_b2.reshape(1, H2),
      o_W3, o_b3.reshape(1, 1))


_MESH = plsc.VectorSubcoreMesh(core_axis_name="c", subcore_axis_name="s")


def _worker_id():
    return lax.axis_index("s") * NC + lax.axis_index("c")


@functools.partial(
    pl.kernel,
    out_type=jax.ShapeDtypeStruct((E_PAD, D), _f32),
    mesh=_MESH,
    scratch_types=[
        pltpu.VMEM((NBLK, KB), _i32),
        pltpu.VMEM((KB, D), _f32),
        pltpu.SemaphoreType.DMA,
    ],
    compiler_params=pltpu.CompilerParams(needs_layout_passes=False),
)
def _sc_gather(m_hbm, src_hbm, out_hbm, idx_v, rows_v, sem):
    wid = _worker_id()
    pltpu.sync_copy(src_hbm.at[wid], idx_v)
    base = wid * EPW
    for b in range(NBLK):
        pltpu.async_copy(m_hbm.at[idx_v.at[b]], rows_v, sem).wait()
        pltpu.sync_copy(rows_v, out_hbm.at[pl.ds(base + b * KB, KB)])


CH = 27
CB = CH * KB


@functools.partial(
    pl.kernel,
    out_type=(
        jax.ShapeDtypeStruct((NC, NP, D), _f32),
        jax.ShapeDtypeStruct((NC, NP), _f32),
    ),
    mesh=_MESH,
    scratch_types=[
        pltpu.VMEM((CH, KB), _i32),
        pltpu.VMEM((CB,), _f32),
        pltpu.VMEM((CH, KB), _f32),
        pltpu.VMEM((KB, D), _f32),
        pltpu.VMEM((NPS,), _f32),
        pltpu.VMEM_SHARED((NP, D), _f32),
        pltpu.VMEM_SHARED((NP,), _f32),
        pltpu.SemaphoreType.DMA,
    ],
    compiler_params=pltpu.CompilerParams(needs_layout_passes=False),
)
def _sc_spmm(msg_hbm, dst_hbm, q_hbm, gexp_hbm, agg_out, den_out,
             dst_v, q_v, g_v, rows_v, sb_v, agg_sh, den_sh, sem):
    cid = lax.axis_index("c")
    sid = lax.axis_index("s")
    wid = _worker_id()

    def _zrow(i, _):
        for ch in range(D // 16):
            rows_v[i, pl.ds(ch * 16, 16)] = jnp.zeros((16,), _f32)
        return 0

    lax.fori_loop(0, KB, _zrow, 0)

    def _zsmall(i, _):
        sb_v[pl.ds(i * 16, 16)] = jnp.zeros((16,), _f32)
        return 0

    lax.fori_loop(0, NPS // 16, _zsmall, 0)

    for r in range(NPS // KB):
        pltpu.sync_copy(rows_v, agg_sh.at[pl.ds(sid * NPS + r * KB, KB)])
    pltpu.sync_copy(sb_v, den_sh.at[pl.ds(sid * NPS, NPS)])
    plsc.subcore_barrier()

    base = wid * EPW
    for c in range(NBLK // CH):
        ci = wid * (NBLK // CH) + c
        pltpu.sync_copy(dst_hbm.at[ci], dst_v)
        pltpu.sync_copy(q_hbm.at[ci], q_v)
        pltpu.sync_copy(gexp_hbm.at[ci], g_v)

        def _block(b, _):
            pltpu.async_copy(
                msg_hbm.at[pl.ds(base + (c * CH + b) * KB, KB)], rows_v,
                sem).wait()

            def _scale(j, _):
                cf = plsc.load_gather(q_v,
                                      [jnp.zeros((16,), _i32) + (b * KB + j)])
                for ch in range(D // 16):
                    s = pl.ds(ch * 16, 16)
                    rows_v[j, s] = rows_v[j, s] * cf
                return 0

            lax.fori_loop(0, KB, _scale, 0)
            pltpu.sync_copy(rows_v, agg_sh.at[dst_v.at[b]], add=True)
            pltpu.sync_copy(g_v.at[b], den_sh.at[dst_v.at[b]], add=True)
            return 0

        lax.fori_loop(0, CH, _block, 0)
    plsc.subcore_barrier()
    for r in range(NPS // KB):
        s = pl.ds(sid * NPS + r * KB, KB)
        pltpu.sync_copy(agg_sh.at[s], rows_v)
        pltpu.sync_copy(rows_v, agg_out.at[cid, s])
    s = pl.ds(sid * NPS, NPS)
    pltpu.sync_copy(den_sh.at[s], sb_v)
    pltpu.sync_copy(sb_v, den_out.at[cid, s])


def kernel(x, edge_attr, W_g, gru_Wih, gru_Whh, gru_bih, gru_bhh, g_W1, g_b1,
           g_W2, g_b2, g_W3, g_b3, ai_W1, ai_b1, ai_W2, ai_b2, aj_W, aj_b,
           o_W1, o_b1, o_W2, o_b2, o_W3, o_b3, edge_index, batch):
    pad_e = E_PAD - (edge_attr.shape[0] + N)
    loop = jnp.arange(N, dtype=_i32)
    zpad = jnp.zeros((pad_e,), _i32)
    src = jnp.concatenate([edge_index[0], loop, zpad])
    dst = jnp.concatenate([edge_index[1], loop, zpad])
    ea = jnp.concatenate([edge_attr, jnp.ones((N,), _f32),
                          jnp.full((pad_e,), 1e9, _f32)])
    src3 = src.reshape(NW, NBLK, KB)
    dst4 = dst.reshape(-1, CH, KB)
    ea_col = ea.reshape(E_PAD, 1)

    xp = jnp.pad(x, ((0, NP - N), (0, 0)))
    batchf = jnp.pad(batch.astype(_f32), (0, NP - N),
                     constant_values=float(NG)).reshape(NP, 1)

    h = xp
    m = _tc_dense0(h, W_g[0])
    for i in range(LAYERS):
        msg = _sc_gather(m, src3)
        gexp, q = _tc_gate(msg, ea_col, g_W1, g_b1, g_W2, g_b2, g_W3, g_b3)
        agg, den = _sc_spmm(msg, dst4, q.reshape(-1, CB),
                            gexp.reshape(-1, CH, KB))
        last = i == LAYERS - 1
        outs = _tc_gru(agg[0], agg[1], den.T, h, gru_Wih, gru_Whh, gru_bih,
                       gru_bhh, W_g[min(i + 1, LAYERS - 1)],
                       with_dense=not last)
        if last:
            h = outs[0]
        else:
            h, m = outs

    return _tc_readout(h, xp, batchf, ai_W1, ai_b1, ai_W2, ai_b2, aj_W,
                       aj_b, o_W1, o_b1, o_W2, o_b2, o_W3, o_b3)


# speedup vs baseline: 8.4602x; 1.1812x over previous
import functools

import jax
import jax.numpy as jnp
from jax import lax
from jax.experimental import pallas as pl
from jax.experimental.pallas import tpu as pltpu
from jax.experimental.pallas import tpu_sc as plsc

N = 10000
D = 128
NG = 64
CUTOFF = 3.5
LAYERS = 4

NC = 2
NS = 16
NW = NC * NS

KB = 64
NBLK = 162
EPW = NBLK * KB
E_PAD = NW * EPW
NBUF = 3
NP = 10240
NPS = NP // NS

_f32 = jnp.float32
_i32 = jnp.int32


def _ss(v):
    return v / (1.0 + jnp.abs(v))


RB = 1024


def _dense0_body(h_ref, Wg_ref, m_ref):
    m_ref[...] = jnp.dot(h_ref[...], Wg_ref[...], preferred_element_type=_f32)


def _tc_dense0(h, Wg):
    return pl.pallas_call(
        _dense0_body,
        grid=(NP // RB,),
        in_specs=[
            pl.BlockSpec((RB, D), lambda i: (i, 0)),
            pl.BlockSpec((D, D), lambda i: (0, 0)),
        ],
        out_specs=pl.BlockSpec((RB, D), lambda i: (i, 0)),
        out_shape=jax.ShapeDtypeStruct((NP, D), _f32),
    )(h, Wg)


def _gru_body(a0_ref, a1_ref, den_ref, h_ref, Wih_ref, Whh_ref, bih_ref,
              bhh_ref, Wg_ref, *out_refs, with_dense):
    den = den_ref[...]
    rden = 1.0 / (den[:, 0:1] + den[:, 1:2] + 1e-16)
    agg = (a0_ref[...] + a1_ref[...]) * rden
    h = h_ref[...]
    gi = jnp.dot(agg, Wih_ref[...].T, preferred_element_type=_f32) + bih_ref[...]
    gh = jnp.dot(h, Whh_ref[...].T, preferred_element_type=_f32) + bhh_ref[...]
    r = jax.nn.sigmoid(gi[:, :D] + gh[:, :D])
    z = jax.nn.sigmoid(gi[:, D:2 * D] + gh[:, D:2 * D])
    n = jnp.tanh(gi[:, 2 * D:] + r * gh[:, 2 * D:])
    h2 = (1.0 - z) * n + z * h
    out_refs[0][...] = h2
    if with_dense:
        out_refs[1][...] = jnp.dot(h2, Wg_ref[...],
                                   preferred_element_type=_f32)


def _tc_gru(a0, a1, denT, h, Wih, Whh, bih, bhh, Wg, with_dense):
    out_specs = [pl.BlockSpec((RB, D), lambda i: (i, 0))]
    out_shape = [jax.ShapeDtypeStruct((NP, D), _f32)]
    if with_dense:
        out_specs.append(pl.BlockSpec((RB, D), lambda i: (i, 0)))
        out_shape.append(jax.ShapeDtypeStruct((NP, D), _f32))
    return pl.pallas_call(
        functools.partial(_gru_body, with_dense=with_dense),
        grid=(NP // RB,),
        in_specs=[
            pl.BlockSpec((RB, D), lambda i: (i, 0)),
            pl.BlockSpec((RB, D), lambda i: (i, 0)),
            pl.BlockSpec((RB, NC), lambda i: (i, 0)),
            pl.BlockSpec((RB, D), lambda i: (i, 0)),
            pl.BlockSpec((3 * D, D), lambda i: (0, 0)),
            pl.BlockSpec((3 * D, D), lambda i: (0, 0)),
            pl.BlockSpec((1, 3 * D), lambda i: (0, 0)),
            pl.BlockSpec((1, 3 * D), lambda i: (0, 0)),
            pl.BlockSpec((D, D), lambda i: (0, 0)),
        ],
        out_specs=out_specs,
        out_shape=out_shape,
    )(a0, a1, denT, h, Wih, Whh, bih.reshape(1, -1), bhh.reshape(1, -1), Wg)


EB = 4096


def _gate_body(msg_ref, ea_ref, W1_ref, b1_ref, W2_ref, b2_ref, w3_ref,
               b3_ref, gexp_ref, q_ref):
    msg = msg_ref[...]
    ea = ea_ref[...]
    mask = ea <= CUTOFF
    ew = jnp.where(mask, ea, 0.0)
    g1 = jnp.dot(msg, W1_ref[...].T, preferred_element_type=_f32)
    t1 = _ss(ew * g1 + b1_ref[...])
    t2 = _ss(jnp.dot(t1, W2_ref[...].T, preferred_element_type=_f32)
             + b2_ref[...])
    gate = jnp.sum(t2 * w3_ref[...], axis=1, keepdims=True) + b3_ref[0, 0]
    gexp = jnp.where(mask, jnp.exp(gate), 0.0)
    gexp_ref[...] = gexp
    q_ref[...] = gexp * ew


def _tc_gate(msg, ea_col, g_W1, g_b1, g_W2, g_b2, g_W3, g_b3):
    return pl.pallas_call(
        _gate_body,
        grid=(E_PAD // EB,),
        in_specs=[
            pl.BlockSpec((EB, D), lambda i: (i, 0)),
            pl.BlockSpec((EB, 1), lambda i: (i, 0)),
            pl.BlockSpec((64, D), lambda i: (0, 0)),
            pl.BlockSpec((1, 64), lambda i: (0, 0)),
            pl.BlockSpec((32, 64), lambda i: (0, 0)),
            pl.BlockSpec((1, 32), lambda i: (0, 0)),
            pl.BlockSpec((1, 32), lambda i: (0, 0)),
            pl.BlockSpec((1, 1), lambda i: (0, 0)),
        ],
        out_specs=[
            pl.BlockSpec((EB, 1), lambda i: (i, 0)),
            pl.BlockSpec((EB, 1), lambda i: (i, 0)),
        ],
        out_shape=[
            jax.ShapeDtypeStruct((E_PAD, 1), _f32),
            jax.ShapeDtypeStruct((E_PAD, 1), _f32),
        ],
    )(msg, ea_col, g_W1, g_b1.reshape(1, 64), g_W2, g_b2.reshape(1, 32),
      g_W3.reshape(1, 32), g_b3.reshape(1, 1))


def _readout_body(h_ref, x_ref, bf_ref, W1h_ref, W1x_ref, b1_ref, W2_ref,
                  b2_ref, Wj_ref, bj_ref, oW1_ref, ob1_ref, oW2_ref, ob2_ref,
                  oW3_ref, ob3_ref, out_ref, acc_ref):
    i = pl.program_id(0)

    @pl.when(i == 0)
    def _():
        acc_ref[...] = jnp.zeros((NG, D), _f32)

    h = h_ref[...]
    x = x_ref[...]
    a1 = _ss(jnp.dot(h, W1h_ref[...].T, preferred_element_type=_f32)
             + jnp.dot(x, W1x_ref[...].T, preferred_element_type=_f32)
             + b1_ref[...])
    ai = _ss(jnp.dot(a1, W2_ref[...].T, preferred_element_type=_f32)
             + b2_ref[...])
    aj = _ss(jnp.dot(x, Wj_ref[...].T, preferred_element_type=_f32)
             + bj_ref[...])
    rowmax = jnp.max(ai, axis=1, keepdims=True)
    e = jnp.exp(ai - rowmax)
    attn = e / jnp.sum(e, axis=1, keepdims=True) * aj
    gids = lax.broadcasted_iota(_i32, (1, NG), 1).astype(_f32)
    oh = (bf_ref[...] == gids).astype(_f32)
    acc_ref[...] += lax.dot_general(oh, attn, (((0,), (0,)), ((), ())),
                                    preferred_element_type=_f32)

    @pl.when(i == pl.num_programs(0) - 1)
    def _():
        pooled = acc_ref[...]
        y = jax.nn.relu(jnp.dot(pooled, oW1_ref[...].T,
                                preferred_element_type=_f32) + ob1_ref[...])
        y = jax.nn.relu(jnp.dot(y, oW2_ref[...].T,
                                preferred_element_type=_f32) + ob2_ref[...])
        out_ref[...] = (jnp.sum(y * oW3_ref[...], axis=1, keepdims=True)
                        + ob3_ref[0, 0])


def _tc_readout(h, x, batchf, ai_W1, ai_b1, ai_W2, ai_b2, aj_W, aj_b,
                o_W1, o_b1, o_W2, o_b2, o_W3, o_b3):
    H1, H2 = o_W1.shape[0], o_W2.shape[0]
    return pl.pallas_call(
        _readout_body,
        grid=(NP // RB,),
        in_specs=[
            pl.BlockSpec((RB, D), lambda i: (i, 0)),
            pl.BlockSpec((RB, D), lambda i: (i, 0)),
            pl.BlockSpec((RB, 1), lambda i: (i, 0)),
            pl.BlockSpec((D, D), lambda i: (0, 0)),
            pl.BlockSpec((D, D), lambda i: (0, 0)),
            pl.BlockSpec((1, D), lambda i: (0, 0)),
            pl.BlockSpec((D, D), lambda i: (0, 0)),
            pl.BlockSpec((1, D), lambda i: (0, 0)),
            pl.BlockSpec((D, D), lambda i: (0, 0)),
            pl.BlockSpec((1, D), lambda i: (0, 0)),
            pl.BlockSpec((H1, D), lambda i: (0, 0)),
            pl.BlockSpec((1, H1), lambda i: (0, 0)),
            pl.BlockSpec((H2, H1), lambda i: (0, 0)),
            pl.BlockSpec((1, H2), lambda i: (0, 0)),
            pl.BlockSpec((1, H2), lambda i: (0, 0)),
            pl.BlockSpec((1, 1), lambda i: (0, 0)),
        ],
        out_specs=pl.BlockSpec((NG, 1), lambda i: (0, 0)),
        out_shape=jax.ShapeDtypeStruct((NG, 1), _f32),
        scratch_shapes=[pltpu.VMEM((NG, D), _f32)],
    )(h, x, batchf, ai_W1[:, :D], ai_W1[:, D:], ai_b1.reshape(1, D),
      ai_W2, ai_b2.reshape(1, D), aj_W, aj_b.reshape(1, D),
      o_W1, o_b1.reshape(1, H1), o_W2, o_b2.reshape(1, H2),
      o_W3, o_b3.reshape(1, 1))


_MESH = plsc.VectorSubcoreMesh(core_axis_name="c", subcore_axis_name="s")


def _worker_id():
    return lax.axis_index("s") * NC + lax.axis_index("c")


@functools.partial(
    pl.kernel,
    out_type=jax.ShapeDtypeStruct((E_PAD, D), _f32),
    mesh=_MESH,
    scratch_types=[
        pltpu.VMEM((NBLK, KB), _i32),
        pltpu.VMEM((NBUF, KB, D), _f32),
    ] + [pltpu.SemaphoreType.DMA] * (2 * NBUF),
    compiler_params=pltpu.CompilerParams(needs_layout_passes=False),
)
def _sc_gather(m_hbm, src_hbm, out_hbm, idx_v, rows_v, *sems):
    gsem, ssem = sems[:NBUF], sems[NBUF:]
    wid = _worker_id()
    pltpu.sync_copy(src_hbm.at[wid], idx_v)
    base = wid * EPW
    for i in range(NBUF):
        pltpu.async_copy(m_hbm.at[idx_v.at[i]], rows_v.at[i], gsem[i])

    def _round(t, _):
        for i in range(NBUF):
            b = t * NBUF + i
            pltpu.make_async_copy(m_hbm.at[idx_v.at[b]], rows_v.at[i],
                                  gsem[i]).wait()
            pltpu.async_copy(rows_v.at[i],
                             out_hbm.at[pl.ds(base + b * KB, KB)], ssem[i])
            nb = b + NBUF

            @pl.when(nb < NBLK)
            def _():
                pltpu.make_async_copy(
                    rows_v.at[i], out_hbm.at[pl.ds(base, KB)],
                    ssem[i]).wait()
                pltpu.async_copy(m_hbm.at[idx_v.at[nb]], rows_v.at[i],
                                 gsem[i])
        return 0

    lax.fori_loop(0, NBLK // NBUF, _round, 0)
    for i in range(NBUF):
        pltpu.make_async_copy(rows_v.at[i], out_hbm.at[pl.ds(base, KB)],
                              ssem[i]).wait()


NCHUNK = 6
CH = NBLK // NCHUNK
CB = CH * KB


@functools.partial(
    pl.kernel,
    out_type=(
        jax.ShapeDtypeStruct((NC, NP, D), _f32),
        jax.ShapeDtypeStruct((NC, NP), _f32),
    ),
    mesh=_MESH,
    scratch_types=[
        pltpu.VMEM((CH, KB), _i32),
        pltpu.VMEM((CB,), _f32),
        pltpu.VMEM((CH, KB), _f32),
        pltpu.VMEM((NBUF, KB, D), _f32),
        pltpu.VMEM((NPS,), _f32),
        pltpu.VMEM_SHARED((NP, D), _f32),
        pltpu.VMEM_SHARED((NP,), _f32),
    ] + [pltpu.SemaphoreType.DMA] * (2 * NBUF),
    compiler_params=pltpu.CompilerParams(needs_layout_passes=False),
)
def _sc_spmm(msg_hbm, dst_hbm, q_hbm, gexp_hbm, agg_out, den_out,
             dst_v, q_v, g_v, rows_v, sb_v, agg_sh, den_sh, *sems):
    rsem, csem = sems[:NBUF], sems[NBUF:]
    cid = lax.axis_index("c")
    sid = lax.axis_index("s")
    wid = _worker_id()

    def _zrow(i, _):
        for ch in range(D // 16):
            rows_v[0, i, pl.ds(ch * 16, 16)] = jnp.zeros((16,), _f32)
        return 0

    lax.fori_loop(0, KB, _zrow, 0)

    def _zsmall(i, _):
        sb_v[pl.ds(i * 16, 16)] = jnp.zeros((16,), _f32)
        return 0

    lax.fori_loop(0, NPS // 16, _zsmall, 0)

    for r in range(NPS // KB):
        pltpu.sync_copy(rows_v.at[0],
                        agg_sh.at[pl.ds(sid * NPS + r * KB, KB)])
    pltpu.sync_copy(sb_v, den_sh.at[pl.ds(sid * NPS, NPS)])
    plsc.subcore_barrier()

    base = wid * EPW
    for c in range(NCHUNK):
        ci = wid * NCHUNK + c
        pltpu.sync_copy(dst_hbm.at[ci], dst_v)
        pltpu.sync_copy(q_hbm.at[ci], q_v)
        pltpu.sync_copy(gexp_hbm.at[ci], g_v)
        cbase = base + c * CB
        for i in range(NBUF):
            pltpu.async_copy(msg_hbm.at[pl.ds(cbase + i * KB, KB)],
                             rows_v.at[i], rsem[i])

        def _round(t, _):
            for i in range(NBUF):
                b = t * NBUF + i
                pltpu.make_async_copy(
                    msg_hbm.at[pl.ds(cbase, KB)], rows_v.at[i],
                    rsem[i]).wait()

                def _scale(j, _):
                    cf = plsc.load_gather(
                        q_v, [jnp.zeros((16,), _i32) + (b * KB + j)])
                    for ch in range(D // 16):
                        s = pl.ds(ch * 16, 16)
                        rows_v[i, j, s] = rows_v[i, j, s] * cf
                    return 0

                lax.fori_loop(0, KB, _scale, 0)
                pltpu.async_copy(rows_v.at[i], agg_sh.at[dst_v.at[b]],
                                 csem[i], add=True)
                pltpu.sync_copy(g_v.at[b], den_sh.at[dst_v.at[b]],
                                add=True)
                nb = b + NBUF

                @pl.when(nb < CH)
                def _():
                    pltpu.make_async_copy(
                        rows_v.at[i], agg_sh.at[dst_v.at[0]],
                        csem[i]).wait()
                    pltpu.async_copy(msg_hbm.at[pl.ds(cbase + nb * KB, KB)],
                                     rows_v.at[i], rsem[i])
            return 0

        lax.fori_loop(0, CH // NBUF, _round, 0)
        for i in range(NBUF):
            pltpu.make_async_copy(rows_v.at[i], agg_sh.at[dst_v.at[0]],
                                  csem[i]).wait()
    plsc.subcore_barrier()
    for r in range(NPS // KB):
        s = pl.ds(sid * NPS + r * KB, KB)
        pltpu.sync_copy(agg_sh.at[s], rows_v.at[0])
        pltpu.sync_copy(rows_v.at[0], agg_out.at[cid, s])
    s = pl.ds(sid * NPS, NPS)
    pltpu.sync_copy(den_sh.at[s], sb_v)
    pltpu.sync_copy(sb_v, den_out.at[cid, s])


def kernel(x, edge_attr, W_g, gru_Wih, gru_Whh, gru_bih, gru_bhh, g_W1, g_b1,
           g_W2, g_b2, g_W3, g_b3, ai_W1, ai_b1, ai_W2, ai_b2, aj_W, aj_b,
           o_W1, o_b1, o_W2, o_b2, o_W3, o_b3, edge_index, batch):
    pad_e = E_PAD - (edge_attr.shape[0] + N)
    loop = jnp.arange(N, dtype=_i32)
    zpad = jnp.zeros((pad_e,), _i32)
    src = jnp.concatenate([edge_index[0], loop, zpad])
    dst = jnp.concatenate([edge_index[1], loop, zpad])
    ea = jnp.concatenate([edge_attr, jnp.ones((N,), _f32),
                          jnp.full((pad_e,), 1e9, _f32)])
    src3 = src.reshape(NW, NBLK, KB)
    dst4 = dst.reshape(-1, CH, KB)
    q_rs = lambda a: a.reshape(-1, CB)
    g_rs = lambda a: a.reshape(-1, CH, KB)
    ea_col = ea.reshape(E_PAD, 1)

    xp = jnp.pad(x, ((0, NP - N), (0, 0)))
    batchf = jnp.pad(batch.astype(_f32), (0, NP - N),
                     constant_values=float(NG)).reshape(NP, 1)

    h = xp
    m = _tc_dense0(h, W_g[0])
    for i in range(LAYERS):
        msg = _sc_gather(m, src3)
        gexp, q = _tc_gate(msg, ea_col, g_W1, g_b1, g_W2, g_b2, g_W3, g_b3)
        agg, den = _sc_spmm(msg, dst4, q_rs(q), g_rs(gexp))
        last = i == LAYERS - 1
        outs = _tc_gru(agg[0], agg[1], den.T, h, gru_Wih, gru_Whh, gru_bih,
                       gru_bhh, W_g[min(i + 1, LAYERS - 1)],
                       with_dense=not last)
        if last:
            h = outs[0]
        else:
            h, m = outs

    return _tc_readout(h, xp, batchf, ai_W1, ai_b1, ai_W2, ai_b2, aj_W,
                       aj_b, o_W1, o_b1, o_W2, o_b2, o_W3, o_b3)


# speedup vs baseline: 10.1965x; 1.2052x over previous
import functools

import jax
import jax.numpy as jnp
from jax import lax
from jax.experimental import pallas as pl
from jax.experimental.pallas import tpu as pltpu
from jax.experimental.pallas import tpu_sc as plsc

N = 10000
D = 128
NG = 64
CUTOFF = 3.5
LAYERS = 4

NC = 2
NS = 16
NW = NC * NS

KB = 64
NBLK = 162
EPW = NBLK * KB
E_PAD = NW * EPW
NBUF = 3
GBUF = 9
E2 = E_PAD // 2
NBLK2 = NBLK // 2
EPW2 = NBLK2 * KB
NP = 10240
NPS = NP // NS

_f32 = jnp.float32
_i32 = jnp.int32


def _ss(v):
    return v / (1.0 + jnp.abs(v))


RB = 1024


def _dense0_body(h_ref, Wg_ref, m_ref):
    m_ref[...] = jnp.dot(h_ref[...], Wg_ref[...], preferred_element_type=_f32)


def _tc_dense0(h, Wg):
    return pl.pallas_call(
        _dense0_body,
        grid=(NP // RB,),
        in_specs=[
            pl.BlockSpec((RB, D), lambda i: (i, 0)),
            pl.BlockSpec((D, D), lambda i: (0, 0)),
        ],
        out_specs=pl.BlockSpec((RB, D), lambda i: (i, 0)),
        out_shape=jax.ShapeDtypeStruct((NP, D), _f32),
    )(h, Wg)


def _gru_body(a0_ref, a1_ref, a2_ref, a3_ref, den_ref, h_ref, Wih_ref,
              Whh_ref, bih_ref, bhh_ref, Wg_ref, *out_refs, with_dense):
    den = den_ref[...]
    rden = 1.0 / (den[:, 0:1] + den[:, 1:2] + den[:, 2:3] + den[:, 3:4]
                  + 1e-16)
    agg = (a0_ref[...] + a1_ref[...] + a2_ref[...] + a3_ref[...]) * rden
    h = h_ref[...]
    gi = jnp.dot(agg, Wih_ref[...].T, preferred_element_type=_f32) + bih_ref[...]
    gh = jnp.dot(h, Whh_ref[...].T, preferred_element_type=_f32) + bhh_ref[...]
    r = jax.nn.sigmoid(gi[:, :D] + gh[:, :D])
    z = jax.nn.sigmoid(gi[:, D:2 * D] + gh[:, D:2 * D])
    n = jnp.tanh(gi[:, 2 * D:] + r * gh[:, 2 * D:])
    h2 = (1.0 - z) * n + z * h
    out_refs[0][...] = h2
    if with_dense:
        out_refs[1][...] = jnp.dot(h2, Wg_ref[...],
                                   preferred_element_type=_f32)


def _tc_gru(a0, a1, a2, a3, denT, h, Wih, Whh, bih, bhh, Wg, with_dense):
    out_specs = [pl.BlockSpec((RB, D), lambda i: (i, 0))]
    out_shape = [jax.ShapeDtypeStruct((NP, D), _f32)]
    if with_dense:
        out_specs.append(pl.BlockSpec((RB, D), lambda i: (i, 0)))
        out_shape.append(jax.ShapeDtypeStruct((NP, D), _f32))
    return pl.pallas_call(
        functools.partial(_gru_body, with_dense=with_dense),
        grid=(NP // RB,),
        in_specs=[
            pl.BlockSpec((RB, D), lambda i: (i, 0)),
            pl.BlockSpec((RB, D), lambda i: (i, 0)),
            pl.BlockSpec((RB, D), lambda i: (i, 0)),
            pl.BlockSpec((RB, D), lambda i: (i, 0)),
            pl.BlockSpec((RB, 2 * NC), lambda i: (i, 0)),
            pl.BlockSpec((RB, D), lambda i: (i, 0)),
            pl.BlockSpec((3 * D, D), lambda i: (0, 0)),
            pl.BlockSpec((3 * D, D), lambda i: (0, 0)),
            pl.BlockSpec((1, 3 * D), lambda i: (0, 0)),
            pl.BlockSpec((1, 3 * D), lambda i: (0, 0)),
            pl.BlockSpec((D, D), lambda i: (0, 0)),
        ],
        out_specs=out_specs,
        out_shape=out_shape,
    )(a0, a1, a2, a3, denT, h, Wih, Whh, bih.reshape(1, -1),
      bhh.reshape(1, -1), Wg)


EB = 5184


def _gate_body(msg_ref, ea_ref, W1_ref, b1_ref, W2_ref, b2_ref, w3_ref,
               b3_ref, gexp_ref, mq_ref):
    msg = msg_ref[...]
    ea = ea_ref[...]
    mask = ea <= CUTOFF
    ew = jnp.where(mask, ea, 0.0)
    g1 = jnp.dot(msg, W1_ref[...].T, preferred_element_type=_f32)
    t1 = _ss(ew * g1 + b1_ref[...])
    t2 = _ss(jnp.dot(t1, W2_ref[...].T, preferred_element_type=_f32)
             + b2_ref[...])
    gate = jnp.sum(t2 * w3_ref[...], axis=1, keepdims=True) + b3_ref[0, 0]
    gexp = jnp.where(mask, jnp.exp(gate), 0.0)
    gexp_ref[...] = gexp
    mq_ref[...] = (gexp * ew) * msg


def _tc_gate(msg, ea_col, g_W1, g_b1, g_W2, g_b2, g_W3, g_b3):
    return pl.pallas_call(
        _gate_body,
        grid=(E2 // EB,),
        in_specs=[
            pl.BlockSpec((EB, D), lambda i: (i, 0)),
            pl.BlockSpec((EB, 1), lambda i: (i, 0)),
            pl.BlockSpec((64, D), lambda i: (0, 0)),
            pl.BlockSpec((1, 64), lambda i: (0, 0)),
            pl.BlockSpec((32, 64), lambda i: (0, 0)),
            pl.BlockSpec((1, 32), lambda i: (0, 0)),
            pl.BlockSpec((1, 32), lambda i: (0, 0)),
            pl.BlockSpec((1, 1), lambda i: (0, 0)),
        ],
        out_specs=[
            pl.BlockSpec((EB, 1), lambda i: (i, 0)),
            pl.BlockSpec((EB, D), lambda i: (i, 0)),
        ],
        out_shape=[
            jax.ShapeDtypeStruct((E2, 1), _f32),
            jax.ShapeDtypeStruct((E2, D), _f32),
        ],
    )(msg, ea_col, g_W1, g_b1.reshape(1, 64), g_W2, g_b2.reshape(1, 32),
      g_W3.reshape(1, 32), g_b3.reshape(1, 1))


def _readout_body(h_ref, x_ref, bf_ref, W1h_ref, W1x_ref, b1_ref, W2_ref,
                  b2_ref, Wj_ref, bj_ref, oW1_ref, ob1_ref, oW2_ref, ob2_ref,
                  oW3_ref, ob3_ref, out_ref, acc_ref):
    i = pl.program_id(0)

    @pl.when(i == 0)
    def _():
        acc_ref[...] = jnp.zeros((NG, D), _f32)

    h = h_ref[...]
    x = x_ref[...]
    a1 = _ss(jnp.dot(h, W1h_ref[...].T, preferred_element_type=_f32)
             + jnp.dot(x, W1x_ref[...].T, preferred_element_type=_f32)
             + b1_ref[...])
    ai = _ss(jnp.dot(a1, W2_ref[...].T, preferred_element_type=_f32)
             + b2_ref[...])
    aj = _ss(jnp.dot(x, Wj_ref[...].T, preferred_element_type=_f32)
             + bj_ref[...])
    rowmax = jnp.max(ai, axis=1, keepdims=True)
    e = jnp.exp(ai - rowmax)
    attn = e / jnp.sum(e, axis=1, keepdims=True) * aj
    gids = lax.broadcasted_iota(_i32, (1, NG), 1).astype(_f32)
    oh = (bf_ref[...] == gids).astype(_f32)
    acc_ref[...] += lax.dot_general(oh, attn, (((0,), (0,)), ((), ())),
                                    preferred_element_type=_f32)

    @pl.when(i == pl.num_programs(0) - 1)
    def _():
        pooled = acc_ref[...]
        y = jax.nn.relu(jnp.dot(pooled, oW1_ref[...].T,
                                preferred_element_type=_f32) + ob1_ref[...])
        y = jax.nn.relu(jnp.dot(y, oW2_ref[...].T,
                                preferred_element_type=_f32) + ob2_ref[...])
        out_ref[...] = (jnp.sum(y * oW3_ref[...], axis=1, keepdims=True)
                        + ob3_ref[0, 0])


def _tc_readout(h, x, batchf, ai_W1, ai_b1, ai_W2, ai_b2, aj_W, aj_b,
                o_W1, o_b1, o_W2, o_b2, o_W3, o_b3):
    H1, H2 = o_W1.shape[0], o_W2.shape[0]
    return pl.pallas_call(
        _readout_body,
        grid=(NP // RB,),
        in_specs=[
            pl.BlockSpec((RB, D), lambda i: (i, 0)),
            pl.BlockSpec((RB, D), lambda i: (i, 0)),
            pl.BlockSpec((RB, 1), lambda i: (i, 0)),
            pl.BlockSpec((D, D), lambda i: (0, 0)),
            pl.BlockSpec((D, D), lambda i: (0, 0)),
            pl.BlockSpec((1, D), lambda i: (0, 0)),
            pl.BlockSpec((D, D), lambda i: (0, 0)),
            pl.BlockSpec((1, D), lambda i: (0, 0)),
            pl.BlockSpec((D, D), lambda i: (0, 0)),
            pl.BlockSpec((1, D), lambda i: (0, 0)),
            pl.BlockSpec((H1, D), lambda i: (0, 0)),
            pl.BlockSpec((1, H1), lambda i: (0, 0)),
            pl.BlockSpec((H2, H1), lambda i: (0, 0)),
            pl.BlockSpec((1, H2), lambda i: (0, 0)),
            pl.BlockSpec((1, H2), lambda i: (0, 0)),
            pl.BlockSpec((1, 1), lambda i: (0, 0)),
        ],
        out_specs=pl.BlockSpec((NG, 1), lambda i: (0, 0)),
        out_shape=jax.ShapeDtypeStruct((NG, 1), _f32),
        scratch_shapes=[pltpu.VMEM((NG, D), _f32)],
    )(h, x, batchf, ai_W1[:, :D], ai_W1[:, D:], ai_b1.reshape(1, D),
      ai_W2, ai_b2.reshape(1, D), aj_W, aj_b.reshape(1, D),
      o_W1, o_b1.reshape(1, H1), o_W2, o_b2.reshape(1, H2),
      o_W3, o_b3.reshape(1, 1))


_MESH = plsc.VectorSubcoreMesh(core_axis_name="c", subcore_axis_name="s")


def _worker_id():
    return lax.axis_index("s") * NC + lax.axis_index("c")


@functools.partial(
    pl.kernel,
    out_type=jax.ShapeDtypeStruct((E2, D), _f32),
    mesh=_MESH,
    scratch_types=[
        pltpu.VMEM((NBLK2, KB), _i32),
        pltpu.VMEM((GBUF, KB, D), _f32),
    ] + [pltpu.SemaphoreType.DMA] * (2 * GBUF),
    compiler_params=pltpu.CompilerParams(needs_layout_passes=False),
)
def _sc_gather(m_hbm, src_hbm, out_hbm, idx_v, rows_v, *sems):
    gsem, ssem = sems[:GBUF], sems[GBUF:]
    wid = _worker_id()
    pltpu.sync_copy(src_hbm.at[wid], idx_v)
    base = wid * EPW2
    for i in range(GBUF):
        pltpu.async_copy(m_hbm.at[idx_v.at[i]], rows_v.at[i], gsem[i])

    def _round(t, _):
        for i in range(GBUF):
            b = t * GBUF + i
            pltpu.make_async_copy(m_hbm.at[idx_v.at[b]], rows_v.at[i],
                                  gsem[i]).wait()
            pltpu.async_copy(rows_v.at[i],
                             out_hbm.at[pl.ds(base + b * KB, KB)], ssem[i])
            nb = b + GBUF

            @pl.when(nb < NBLK2)
            def _():
                pltpu.make_async_copy(
                    rows_v.at[i], out_hbm.at[pl.ds(base, KB)],
                    ssem[i]).wait()
                pltpu.async_copy(m_hbm.at[idx_v.at[nb]], rows_v.at[i],
                                 gsem[i])
        return 0

    lax.fori_loop(0, NBLK2 // GBUF, _round, 0)
    for i in range(GBUF):
        pltpu.make_async_copy(rows_v.at[i], out_hbm.at[pl.ds(base, KB)],
                              ssem[i]).wait()


NCHUNK = 3
CH = NBLK2 // NCHUNK
CB = CH * KB


@functools.partial(
    pl.kernel,
    out_type=(
        jax.ShapeDtypeStruct((NC, NP, D), _f32),
        jax.ShapeDtypeStruct((NC, NP), _f32),
    ),
    mesh=_MESH,
    scratch_types=[
        pltpu.VMEM((CH, KB), _i32),
        pltpu.VMEM((CH, KB), _f32),
        pltpu.VMEM((NBUF, KB, D), _f32),
        pltpu.VMEM((NPS,), _f32),
        pltpu.VMEM_SHARED((NP, D), _f32),
        pltpu.VMEM_SHARED((NP,), _f32),
    ] + [pltpu.SemaphoreType.DMA] * (2 * NBUF),
    compiler_params=pltpu.CompilerParams(needs_layout_passes=False),
)
def _sc_spmm(mq_hbm, dst_hbm, gexp_hbm, agg_out, den_out,
             dst_v, g_v, rows_v, sb_v, agg_sh, den_sh, *sems):
    rsem, csem = sems[:NBUF], sems[NBUF:]
    cid = lax.axis_index("c")
    sid = lax.axis_index("s")
    wid = _worker_id()

    def _zrow(i, _):
        for ch in range(D // 16):
            rows_v[0, i, pl.ds(ch * 16, 16)] = jnp.zeros((16,), _f32)
        return 0

    lax.fori_loop(0, KB, _zrow, 0)

    def _zsmall(i, _):
        sb_v[pl.ds(i * 16, 16)] = jnp.zeros((16,), _f32)
        return 0

    lax.fori_loop(0, NPS // 16, _zsmall, 0)

    for r in range(NPS // KB):
        pltpu.sync_copy(rows_v.at[0],
                        agg_sh.at[pl.ds(sid * NPS + r * KB, KB)])
    pltpu.sync_copy(sb_v, den_sh.at[pl.ds(sid * NPS, NPS)])
    plsc.subcore_barrier()

    base = wid * EPW2
    for c in range(NCHUNK):
        ci = wid * NCHUNK + c
        pltpu.sync_copy(dst_hbm.at[ci], dst_v)
        pltpu.sync_copy(gexp_hbm.at[ci], g_v)
        cbase = base + c * CB
        for i in range(NBUF):
            pltpu.async_copy(mq_hbm.at[pl.ds(cbase + i * KB, KB)],
                             rows_v.at[i], rsem[i])

        def _round(t, _):
            for i in range(NBUF):
                b = t * NBUF + i
                pltpu.make_async_copy(
                    mq_hbm.at[pl.ds(cbase, KB)], rows_v.at[i],
                    rsem[i]).wait()
                pltpu.async_copy(rows_v.at[i], agg_sh.at[dst_v.at[b]],
                                 csem[i], add=True)
                pltpu.sync_copy(g_v.at[b], den_sh.at[dst_v.at[b]],
                                add=True)
                nb = b + NBUF

                @pl.when(nb < CH)
                def _():
                    pltpu.make_async_copy(
                        rows_v.at[i], agg_sh.at[dst_v.at[0]],
                        csem[i]).wait()
                    pltpu.async_copy(mq_hbm.at[pl.ds(cbase + nb * KB, KB)],
                                     rows_v.at[i], rsem[i])
            return 0

        lax.fori_loop(0, CH // NBUF, _round, 0)
        for i in range(NBUF):
            pltpu.make_async_copy(rows_v.at[i], agg_sh.at[dst_v.at[0]],
                                  csem[i]).wait()
    plsc.subcore_barrier()
    for r in range(NPS // KB):
        s = pl.ds(sid * NPS + r * KB, KB)
        pltpu.sync_copy(agg_sh.at[s], rows_v.at[0])
        pltpu.sync_copy(rows_v.at[0], agg_out.at[cid, s])
    s = pl.ds(sid * NPS, NPS)
    pltpu.sync_copy(den_sh.at[s], sb_v)
    pltpu.sync_copy(sb_v, den_out.at[cid, s])


def kernel(x, edge_attr, W_g, gru_Wih, gru_Whh, gru_bih, gru_bhh, g_W1, g_b1,
           g_W2, g_b2, g_W3, g_b3, ai_W1, ai_b1, ai_W2, ai_b2, aj_W, aj_b,
           o_W1, o_b1, o_W2, o_b2, o_W3, o_b3, edge_index, batch):
    pad_e = E_PAD - (edge_attr.shape[0] + N)
    loop = jnp.arange(N, dtype=_i32)
    zpad = jnp.zeros((pad_e,), _i32)
    src = jnp.concatenate([edge_index[0], loop, zpad])
    dst = jnp.concatenate([edge_index[1], loop, zpad])
    ea = jnp.concatenate([edge_attr, jnp.ones((N,), _f32),
                          jnp.full((pad_e,), 1e9, _f32)])
    src_h = [src[:E2].reshape(NW, NBLK2, KB), src[E2:].reshape(NW, NBLK2, KB)]
    dst_h = [dst[:E2].reshape(-1, CH, KB), dst[E2:].reshape(-1, CH, KB)]
    ea_h = [ea[:E2].reshape(E2, 1), ea[E2:].reshape(E2, 1)]
    q_rs = lambda a: a.reshape(-1, CB)
    g_rs = lambda a: a.reshape(-1, CH, KB)

    xp = jnp.pad(x, ((0, NP - N), (0, 0)))
    batchf = jnp.pad(batch.astype(_f32), (0, NP - N),
                     constant_values=float(NG)).reshape(NP, 1)

    h = xp
    m = _tc_dense0(h, W_g[0])
    for i in range(LAYERS):
        msg0 = _sc_gather(m, src_h[0])
        gexp0, mq0 = _tc_gate(msg0, ea_h[0], g_W1, g_b1, g_W2, g_b2, g_W3,
                              g_b3)
        msg1 = _sc_gather(m, src_h[1])
        agg0, den0 = _sc_spmm(mq0, dst_h[0], g_rs(gexp0))
        gexp1, mq1 = _tc_gate(msg1, ea_h[1], g_W1, g_b1, g_W2, g_b2, g_W3,
                              g_b3)
        agg1, den1 = _sc_spmm(mq1, dst_h[1], g_rs(gexp1))
        last = i == LAYERS - 1
        denT = jnp.concatenate([den0.T, den1.T], axis=1)
        outs = _tc_gru(agg0[0], agg0[1], agg1[0], agg1[1], denT, h,
                       gru_Wih, gru_Whh, gru_bih, gru_bhh,
                       W_g[min(i + 1, LAYERS - 1)], with_dense=not last)
        if last:
            h = outs[0]
        else:
            h, m = outs

    return _tc_readout(h, xp, batchf, ai_W1, ai_b1, ai_W2, ai_b2, aj_W,
                       aj_b, o_W1, o_b1, o_W2, o_b2, o_W3, o_b3)
